# baseline (speedup 1.0000x reference)
import functools

import jax
import jax.numpy as jnp
from jax import lax
from jax.experimental import pallas as pl
from jax.experimental.pallas import tpu as pltpu
from jax.experimental.pallas import tpu_sc as plsc

B, C, H, W = 32, 768, 32, 32
N = H * W
R = B * C
K = 205
ALPHA = 0.7

NC, NS, L = 2, 16, 16
NW = NC * NS
RPW = R // NW
GROUP = 16
NGRP = RPW // GROUP
CHUNKS = N // L
NB = 256

MIN32 = -2147483648


def _kernel_body(x_hbm, out_hbm, xbuf, kT, xT, h1c, h1s, h2t, h2b, outbuf):
    wid = lax.axis_index("s") * NC + lax.axis_index("c")
    lanes = lax.iota(jnp.int32, L)
    iota16 = lanes * L
    rowoff = lanes * NB
    zero = jnp.zeros((L,), jnp.int32)
    one = jnp.ones((L,), jnp.int32)
    fzero = jnp.zeros((L,), jnp.float32)
    Kv = jnp.full((L,), K, jnp.int32)

    def zbody(j, _):
        for u in range(4):
            off = (j * 4 + u) * L
            h1c[pl.ds(off, L)] = zero
            h1s[pl.ds(off, L)] = fzero
            h2t[pl.ds(off, L)] = zero
            h2b[pl.ds(off, L)] = zero
        return 0

    lax.fori_loop(0, (GROUP * NB) // (4 * L), zbody, 0)

    def val16(b1, b):
        key16 = (b1 << 8) + b
        ks = ((key16 - 32768) << 16) + 32768
        bv = jnp.where(ks >= 0, ks, MIN32 - ks)
        return plsc.bitcast(bv, jnp.float32)

    def group_body(g, carry):
        row0 = wid * RPW + g * GROUP
        pltpu.sync_copy(x_hbm.at[pl.ds(row0 * N, GROUP * N)], xbuf)

        def p0r(r, _):
            def p0j(j, c):
                for u in range(4):
                    jj = j * 4 + u
                    v = xbuf[pl.ds(r * N + jj * L, L)]
                    b = plsc.bitcast(v, jnp.int32)
                    ks = jnp.where(b >= 0, b, MIN32 - b)
                    k16 = (ks >> 16) + 32768
                    idx = iota16 + (jj * NB + r)
                    plsc.store_scatter(kT, [idx], k16)
                    plsc.store_scatter(xT, [idx], v)
                return c

            return lax.fori_loop(0, CHUNKS // 4, p0j, 0)

        lax.fori_loop(0, GROUP, p0r, 0)

        def l1(j, c):
            for u in range(8):
                e = j * 8 + u
                v = kT[pl.ds(e * L, L)]
                xv = xT[pl.ds(e * L, L)]
                addr = rowoff + (v >> 8)
                plsc.addupdate_scatter(h1c, [addr], one)
                plsc.addupdate_scatter(h1s, [addr], xv)
            return c

        lax.fori_loop(0, N // 8, l1, 0)

        def s1t(j, st):
            c, b1, na, sa = st
            for u in range(8):
                b = 255 - (j * 8 + u)
                col = rowoff + b
                cb = plsc.load_gather(h1c, [col])
                sb = plsc.load_gather(h1s, [col])
                cn = c + cb
                still = cn < K
                na = na + jnp.where(still, cb, zero)
                sa = sa + jnp.where(still, sb, fzero)
                b1 = jnp.where((c < K) & (cn >= K), b, b1)
                c = cn
            return c, b1, na, sa

        _, b1t, na1, sa1 = lax.fori_loop(
            0, NB // 8, s1t, (zero, zero, zero, fzero))

        def s1b(j, st):
            c, b1, na, sa = st
            for u in range(8):
                b = j * 8 + u
                col = rowoff + b
                cb = plsc.load_gather(h1c, [col])
                sb = plsc.load_gather(h1s, [col])
                plsc.store_scatter(h1c, [col], zero)
                plsc.store_scatter(h1s, [col], fzero)
                cn = c + cb
                still = cn < K
                na = na + jnp.where(still, cb, zero)
                sa = sa + jnp.where(still, sb, fzero)
                b1 = jnp.where((c < K) & (cn >= K), b, b1)
                c = cn
            return c, b1, na, sa

        _, b1b, nb1, sb1 = lax.fori_loop(
            0, NB // 8, s1b, (zero, zero, zero, fzero))

        def l2(j, c):
            for u in range(8):
                e = j * 8 + u
                v = kT[pl.ds(e * L, L)]
                hi = v >> 8
                addr = rowoff + (v & 255)
                plsc.addupdate_scatter(h2t, [addr], one, mask=hi == b1t)
                plsc.addupdate_scatter(h2b, [addr], one, mask=hi == b1b)
            return c

        lax.fori_loop(0, N // 8, l2, 0)

        def s2t(j, st):
            c, na, sa, vt = st
            for u in range(8):
                b = 255 - (j * 8 + u)
                col = rowoff + b
                cb = plsc.load_gather(h2t, [col])
                plsc.store_scatter(h2t, [col], zero)
                vv = val16(b1t, b)
                cn = c + cb
                still = cn < K
                na = na + jnp.where(still, cb, zero)
                sa = sa + jnp.where(still, cb.astype(jnp.float32) * vv, fzero)
                vt = jnp.where((c < K) & (cn >= K), vv, vt)
                c = cn
            return c, na, sa, vt

        _, na2, sa2, vtop = lax.fori_loop(
            0, NB // 8, s2t, (na1, zero, fzero, fzero))

        def s2b(j, st):
            c, na, sa, vt = st
            for u in range(8):
                b = j * 8 + u
                col = rowoff + b
                cb = plsc.load_gather(h2b, [col])
                plsc.store_scatter(h2b, [col], zero)
                vv = val16(b1b, b)
                cn = c + cb
                still = cn < K
                na = na + jnp.where(still, cb, zero)
                sa = sa + jnp.where(still, cb.astype(jnp.float32) * vv, fzero)
                vt = jnp.where((c < K) & (cn >= K), vv, vt)
                c = cn
            return c, na, sa, vt

        _, nb2, sb2, vbot = lax.fori_loop(
            0, NB // 8, s2b, (nb1, zero, fzero, fzero))

        remt = (Kv - na1 - na2).astype(jnp.float32)
        remb = (Kv - nb1 - nb2).astype(jnp.float32)
        top_sum = sa1 + sa2 + remt * vtop
        bot_sum = sb1 + sb2 + remb * vbot
        outv = top_sum * (1.0 / (2 * K)) + bot_sum * (ALPHA / (2 * K))
        outbuf[pl.ds(g * GROUP, GROUP)] = outv
        return carry

    lax.fori_loop(0, NGRP, group_body, 0)
    pltpu.sync_copy(outbuf, out_hbm.at[pl.ds(wid * RPW, RPW)])


@jax.jit
def kernel(input):
    x = input.reshape(R * N)
    mesh = plsc.VectorSubcoreMesh(
        core_axis_name="c", subcore_axis_name="s",
        num_cores=NC, num_subcores=NS)
    out = pl.kernel(
        _kernel_body,
        out_type=jax.ShapeDtypeStruct((R,), jnp.float32),
        mesh=mesh,
        compiler_params=pltpu.CompilerParams(needs_layout_passes=False),
        scratch_types=[
            pltpu.VMEM((GROUP * N,), jnp.float32),
            pltpu.VMEM((GROUP * N,), jnp.int32),
            pltpu.VMEM((GROUP * N,), jnp.float32),
            pltpu.VMEM((GROUP * NB,), jnp.int32),
            pltpu.VMEM((GROUP * NB,), jnp.float32),
            pltpu.VMEM((GROUP * NB,), jnp.int32),
            pltpu.VMEM((GROUP * NB,), jnp.int32),
            pltpu.VMEM((RPW,), jnp.float32),
        ],
    )(x)
    return out.reshape(B, C)


# speedup vs baseline: 2.1194x; 2.1194x over previous
import functools

import jax
import jax.numpy as jnp
from jax import lax
from jax.experimental import pallas as pl
from jax.experimental.pallas import tpu as pltpu
from jax.experimental.pallas import tpu_sc as plsc

B, C, H, W = 32, 768, 32, 32
N = H * W
R = B * C
K = 205
ALPHA = 0.7

NC, NS, L = 2, 16, 16
NW = NC * NS
RPW = R // NW
GROUP = 16
NGRP = RPW // GROUP
CH32 = N // (2 * L)


def _kernel_body(x_hbm, out_hbm, xbuf, kbuf, outbuf):
    wid = lax.axis_index("s") * NC + lax.axis_index("c")
    zero = jnp.zeros((L,), jnp.int32)
    one = jnp.ones((L,), jnp.int32)
    fzero = jnp.zeros((L,), jnp.float32)
    bzero = jnp.zeros((2 * L,), jnp.bfloat16)
    bone = jnp.ones((2 * L,), jnp.bfloat16)
    lanes = lax.iota(jnp.int32, L)

    def u2bits(u):
        return jnp.where(u >= 32768, u - 32768, 65535 - u)

    def u2f32vec(u):
        return plsc.bitcast(jnp.full((L,), u2bits(u) << 16, jnp.int32),
                            jnp.float32)

    def u2bfvec(u):
        b = u2bits(u)
        return plsc.bitcast(jnp.full((L,), b | (b << 16), jnp.int32),
                            jnp.bfloat16)

    def group_body(g, carry):
        row0 = wid * RPW + g * GROUP
        pltpu.sync_copy(x_hbm.at[pl.ds(row0 * N, GROUP * N)], xbuf)

        def key_body(j, c):
            for u in range(4):
                off = j * (8 * L) + u * (2 * L)
                a = xbuf[pl.ds(off, L)]
                b = xbuf[pl.ds(off + L, L)]
                p = plsc.pack(a, b, format=plsc.PackFormat.INTERLEAVED)
                kbuf[pl.ds(off // 2, L)] = plsc.bitcast(p, jnp.int32)
            return c

        lax.fori_loop(0, GROUP * N // (8 * L), key_body, 0)

        def row_body(r, ovec):
            base = r * N

            def count_pass(cand1, cand2p):
                cv1 = u2bfvec(cand1)
                cv2 = u2bfvec(cand2p)

                def cbody(j, c):
                    c1a, c1b, c2a, c2b = c
                    for u in range(8):
                        v = plsc.bitcast(
                            kbuf[pl.ds((base + (j * 8 + u) * 2 * L) // 2, L)],
                            jnp.bfloat16)
                        i1 = jnp.where(v >= cv1, bone, bzero)
                        i2 = jnp.where(v <= cv2, bone, bzero)
                        if u % 2 == 0:
                            c1a = c1a + i1
                            c2a = c2a + i2
                        else:
                            c1b = c1b + i1
                            c2b = c2b + i2
                    return c1a, c1b, c2a, c2b

                c1a, c1b, c2a, c2b = lax.fori_loop(
                    0, CH32 // 8, cbody, (bzero, bzero, bzero, bzero))
                u1a, u1b = plsc.unpack(c1a + c1b,
                                       format=plsc.PackFormat.INTERLEAVED)
                u2a, u2b = plsc.unpack(c2a + c2b,
                                       format=plsc.PackFormat.INTERLEAVED)
                return jnp.sum(u1a + u1b), jnp.sum(u2a + u2b)

            def bit_body(i, st):
                t1, t2, bit = st
                cand1 = t1 + bit
                cand2 = t2 + bit
                n1, n2 = count_pass(cand1, 65535 - cand2)
                t1 = jnp.where(n1 >= float(K), cand1, t1)
                t2 = jnp.where(n2 >= float(K), cand2, t2)
                return t1, t2, bit >> 1

            t1, t2, _ = lax.fori_loop(
                0, 16, bit_body,
                (jnp.int32(0), jnp.int32(0), jnp.int32(32768)))
            thr_bot = 65535 - t2

            val_top = u2f32vec(t1)
            val_bot = u2f32vec(thr_bot)
            ub = 0.5 * (val_top + u2f32vec(t1 + 1))
            lb = 0.5 * (val_bot + u2f32vec(thr_bot - 1))

            def fbody(j, c):
                cg, sg, cl, sl = c
                for u in range(8):
                    xv = xbuf[pl.ds(base + (j * 8 + u) * L, L)]
                    m1 = xv > ub
                    m2 = xv < lb
                    cg = cg + jnp.where(m1, one, zero)
                    sg = sg + jnp.where(m1, xv, fzero)
                    cl = cl + jnp.where(m2, one, zero)
                    sl = sl + jnp.where(m2, xv, fzero)
                return cg, sg, cl, sl

            cg, sg, cl, sl = lax.fori_loop(
                0, N // (8 * L), fbody, (zero, fzero, zero, fzero))

            ng = jnp.full((L,), K - jnp.sum(cg), jnp.int32).astype(jnp.float32)
            nl = jnp.full((L,), K - jnp.sum(cl), jnp.int32).astype(jnp.float32)
            sgv = jnp.full((L,), jnp.sum(sg), jnp.float32)
            slv = jnp.full((L,), jnp.sum(sl), jnp.float32)
            top_sum = sgv + ng * val_top
            bot_sum = slv + nl * val_bot
            outv = top_sum * (1.0 / (2 * K)) + bot_sum * (ALPHA / (2 * K))
            return jnp.where(lanes == r, outv, ovec)

        ovec = lax.fori_loop(0, GROUP, row_body, fzero)
        outbuf[pl.ds(g * GROUP, GROUP)] = ovec
        return carry

    lax.fori_loop(0, NGRP, group_body, 0)
    pltpu.sync_copy(outbuf, out_hbm.at[pl.ds(wid * RPW, RPW)])


@jax.jit
def kernel(input):
    x = input.reshape(R * N)
    mesh = plsc.VectorSubcoreMesh(
        core_axis_name="c", subcore_axis_name="s",
        num_cores=NC, num_subcores=NS)
    out = pl.kernel(
        _kernel_body,
        out_type=jax.ShapeDtypeStruct((R,), jnp.float32),
        mesh=mesh,
        compiler_params=pltpu.CompilerParams(needs_layout_passes=False),
        scratch_types=[
            pltpu.VMEM((GROUP * N,), jnp.float32),
            pltpu.VMEM((GROUP * N // 2,), jnp.int32),
            pltpu.VMEM((RPW,), jnp.float32),
        ],
    )(x)
    return out.reshape(B, C)
